# baseline (speedup 1.0000x reference)
import jax
import jax.numpy as jnp
from jax import lax
from jax.experimental import pallas as pl
from jax.experimental.pallas import tpu as pltpu

_C = 64
_P = 16
_D = _C + _P
_K = 16
_BRANCH = 2
_NCLOUD = 8
_I = 1024
_LR = 0.001
_RB = 256
_N = _NCLOUD * _I
_KG = _NCLOUD * _K


def _labels_body(fmb_ref, wmlp_ref, bmlp_ref, x_ref, oh_ref, oh_scr):
    f32 = jnp.float32
    bf16 = jnp.bfloat16
    hp = lax.Precision.HIGHEST
    fmb = fmb_ref[...].reshape(_N, _C)
    mm = jnp.dot(fmb.astype(bf16), wmlp_ref[...].astype(bf16),
                 preferred_element_type=f32)
    x2 = fmb + _LR * jnp.maximum(mm + bmlp_ref[...], 0.0)
    x_ref[...] = x2.reshape(_NCLOUD, _I, _C)

    centT0 = jnp.concatenate(
        [jnp.transpose(x2[g * _I:g * _I + _K, :]) for g in range(_NCLOUD)],
        axis=1)
    col = lax.broadcasted_iota(jnp.int32, (_I, _KG), 1)
    colg = col // _K
    big = jnp.float32(1e30)

    def outer(it, centT):
        c2row = jnp.sum(centT * centT, axis=0, keepdims=True)

        def inner(g, carry):
            sat, cntr = carry
            xc = x_ref[g]
            s = jnp.dot(xc, centT, precision=hp)
            d = jnp.where(colg == g, c2row - 2.0 * s, big)
            dmin = jnp.min(d, axis=1, keepdims=True)
            lab = jnp.min(jnp.where(d == dmin, col, _KG), axis=1,
                          keepdims=True)
            ohc = (col == lab).astype(f32)
            oh_scr[pl.ds(g * _I, _I), :] = ohc
            sat = sat + lax.dot_general(
                xc.astype(bf16), ohc.astype(bf16),
                (((0,), (0,)), ((), ())), preferred_element_type=f32)
            cntr = cntr + jnp.sum(ohc, axis=0, keepdims=True)
            return (sat, cntr)

        sat, cntr = lax.fori_loop(
            0, _NCLOUD, inner,
            (jnp.zeros((_C, _KG), f32), jnp.zeros((1, _KG), f32)))
        return jnp.where(cntr > 0.0, sat / jnp.maximum(cntr, 1.0), centT)

    lax.fori_loop(0, 5, outer, centT0)
    for g in range(_NCLOUD):
        oh_ref[g] = oh_scr[g * _I:(g + 1) * _I, g * _K:(g + 1) * _K]


def _cloud_body(x_ref, oh_ref, tfw_ref, bfw_ref, wm_row_ref,
                bm_ref, wb_ref, bb_ref, wsig_ref, bsig_ref, wm1_ref, bm1_ref,
                wm2_ref, bm2_ref, wm3_ref, bm3_ref, out_ref):
    f32 = jnp.float32
    bf16 = jnp.bfloat16
    x = x_ref[0]
    oh = oh_ref[0]

    x16 = x.astype(bf16)
    hm_blocks = []
    for rb in range(_I // _RB):
        xb = x[rb * _RB:(rb + 1) * _RB, :]
        xb16 = x16[rb * _RB:(rb + 1) * _RB, :]
        xx = jnp.concatenate(
            [xb16[:, c:] * xb16[:, c:c + 1] for c in range(_C)], axis=1)
        hm_blocks.append(
            jnp.dot(xx, tfw_ref[...], preferred_element_type=f32)
            + xb @ bfw_ref[...])
    hm = jnp.concatenate(hm_blocks, axis=0)
    hd = hm[:, :_D]
    msg = hm[:, _D:]

    gate = jax.nn.sigmoid(
        jnp.sum(msg * wm_row_ref[...], axis=1, keepdims=True) + bm_ref[...])
    gm = gate * msg
    seg = lax.dot_general(oh, gm, (((0,), (0,)), ((), ())))
    cnt = jnp.sum(oh, axis=0)[:, None]
    agg = seg / jnp.maximum(cnt, 1.0)
    h = hd + oh @ agg

    t = h + h @ wb_ref[...] + bb_ref[...]
    conv = jnp.maximum(t @ wsig_ref[...] + bsig_ref[...], 0.0)

    fm = jnp.concatenate([x, jnp.zeros((_I, _P), f32)], axis=1)
    sg = jnp.sum(fm + conv, axis=0, keepdims=True) * (1.0 / _I)
    a1 = jnp.maximum(sg @ wm1_ref[...] + bm1_ref[...], 0.0)
    a2 = jnp.maximum(sg @ wm2_ref[...] + bm2_ref[...], 0.0)
    mx = jnp.maximum(a1, a2)
    e1 = jnp.exp(a1 - mx)
    e2 = jnp.exp(a2 - mx)
    den = e1 + e2
    fused = (e1 / den) * fm + (e2 / den) * conv
    out_ref[0, :_I, :] = fused

    neg = jnp.float32(-jnp.inf)
    enc_rows = []
    for k in range(_K):
        mk = oh[:, k:k + 1] > 0.0
        enc_rows.append(
            jnp.max(jnp.where(mk, fused, neg), axis=0, keepdims=True))
    enc = jnp.concatenate(enc_rows, axis=0)
    enc = jnp.where(cnt > 0.0, enc, 0.0)

    ups = enc @ wm3_ref[...] + bm3_ref[...]
    for k in range(_K):
        out_ref[0, _I + 2 * k, :] = ups[k, :_D]
        out_ref[0, _I + 2 * k + 1, :] = ups[k, _D:]


def kernel(feature_matrix_batch, Wmlp, bmlp, Wf, bf, Ww, bw, Wm, bm, Wb, bb,
           Wsig, bsig, Wm1, bm1, Wm2, bm2, Wm3, bm3):
    f32 = jnp.float32

    def fixed(shape):
        return pl.BlockSpec(shape, lambda i: (0,) * len(shape))

    x_all, oh_all = pl.pallas_call(
        _labels_body,
        in_specs=[
            pl.BlockSpec((_NCLOUD, _I, _C), lambda: (0, 0, 0)),
            pl.BlockSpec((_C, _C), lambda: (0, 0)),
            pl.BlockSpec((1, _C), lambda: (0, 0)),
        ],
        out_specs=[
            pl.BlockSpec((_NCLOUD, _I, _C), lambda: (0, 0, 0)),
            pl.BlockSpec((_NCLOUD, _I, _K), lambda: (0, 0, 0)),
        ],
        out_shape=[
            jax.ShapeDtypeStruct((_NCLOUD, _I, _C), f32),
            jax.ShapeDtypeStruct((_NCLOUD, _I, _K), f32),
        ],
        scratch_shapes=[pltpu.VMEM((_N, _KG), f32)],
    )(feature_matrix_batch, Wmlp, bmlp.reshape(1, _C))

    def symfold(W):
        W3 = W.reshape(_C, _C, _D)
        Wsum = W3 + W3.transpose(1, 0, 2)
        Whalf = Wsum - W3 * jnp.eye(_C, dtype=W.dtype)[:, :, None]
        return jnp.concatenate([Whalf[c, c:, :] for c in range(_C)], axis=0)

    tfw = jnp.concatenate(
        [symfold(Wf), symfold(Ww)], axis=1).astype(jnp.bfloat16)
    bfw = jnp.concatenate([bf.reshape(_C, _D), bw.reshape(_C, _D)], axis=1)
    args = (x_all, oh_all, tfw, bfw, Wm.reshape(1, _D), bm.reshape(1, 1),
            Wb, bb.reshape(1, _D), Wsig, bsig.reshape(1, _D),
            Wm1, bm1.reshape(1, _D), Wm2, bm2.reshape(1, _D),
            Wm3, bm3.reshape(1, _BRANCH * _D))
    in_specs = [
        pl.BlockSpec((1, _I, _C), lambda i: (i, 0, 0)),
        pl.BlockSpec((1, _I, _K), lambda i: (i, 0, 0)),
        fixed((_C * (_C + 1) // 2, 2 * _D)), fixed((_C, 2 * _D)),
        fixed((1, _D)), fixed((1, 1)),
        fixed((_D, _D)), fixed((1, _D)),
        fixed((_D, _D)), fixed((1, _D)),
        fixed((_D, _D)), fixed((1, _D)),
        fixed((_D, _D)), fixed((1, _D)),
        fixed((_D, _BRANCH * _D)), fixed((1, _BRANCH * _D)),
    ]
    nrows = _I + _BRANCH * _K
    return pl.pallas_call(
        _cloud_body,
        grid=(_NCLOUD,),
        in_specs=in_specs,
        out_specs=pl.BlockSpec((1, nrows, _D), lambda i: (i, 0, 0)),
        out_shape=jax.ShapeDtypeStruct((_NCLOUD, nrows, _D), f32),
    )(*args)


# speedup vs baseline: 1.0821x; 1.0821x over previous
import jax
import jax.numpy as jnp
from jax import lax
from jax.experimental import pallas as pl
from jax.experimental.pallas import tpu as pltpu

_C = 64
_P = 16
_D = _C + _P
_K = 16
_BRANCH = 2
_NCLOUD = 8
_I = 1024
_LR = 0.001
_RB = 256
_N = _NCLOUD * _I
_KG = _NCLOUD * _K


def _labels_body(fmb_ref, wmlp_ref, bmlp_ref, x_ref, oh_ref, oh_scr):
    f32 = jnp.float32
    bf16 = jnp.bfloat16
    hp = lax.Precision.HIGHEST
    fmb = fmb_ref[...].reshape(_N, _C)
    mm = jnp.dot(fmb.astype(bf16), wmlp_ref[...].astype(bf16),
                 preferred_element_type=f32)
    x2 = fmb + _LR * jnp.maximum(mm + bmlp_ref[...], 0.0)
    x_ref[...] = x2.reshape(_NCLOUD, _I, _C)

    centT0 = jnp.concatenate(
        [jnp.transpose(x2[g * _I:g * _I + _K, :]) for g in range(_NCLOUD)],
        axis=1)
    col = lax.broadcasted_iota(jnp.int32, (_I, _KG), 1)
    colg = col // _K
    big = jnp.float32(1e30)

    def outer(it, centT):
        c2row = jnp.sum(centT * centT, axis=0, keepdims=True)

        def inner(g, carry):
            sat, cntr = carry
            xc = x_ref[g]
            s = jnp.dot(xc, centT, precision=hp)
            d = jnp.where(colg == g, c2row - 2.0 * s, big)
            dmin = jnp.min(d, axis=1, keepdims=True)
            lab = jnp.min(jnp.where(d == dmin, col, _KG), axis=1,
                          keepdims=True)
            ohc = (col == lab).astype(f32)
            oh_scr[pl.ds(g * _I, _I), :] = ohc
            sat = sat + lax.dot_general(
                xc.astype(bf16), ohc.astype(bf16),
                (((0,), (0,)), ((), ())), preferred_element_type=f32)
            cntr = cntr + jnp.sum(ohc, axis=0, keepdims=True)
            return (sat, cntr)

        sat, cntr = lax.fori_loop(
            0, _NCLOUD, inner,
            (jnp.zeros((_C, _KG), f32), jnp.zeros((1, _KG), f32)))
        return jnp.where(cntr > 0.0, sat / jnp.maximum(cntr, 1.0), centT)

    lax.fori_loop(0, 5, outer, centT0)
    for g in range(_NCLOUD):
        oh_ref[g] = oh_scr[g * _I:(g + 1) * _I, g * _K:(g + 1) * _K]


def _cloud_body(x_ref, oh_ref, tfw_ref, bfw_ref, wm_row_ref,
                bm_ref, wb_ref, bb_ref, wsig_ref, bsig_ref, wm1_ref, bm1_ref,
                wm2_ref, bm2_ref, wm3_ref, bm3_ref, out_ref):
    f32 = jnp.float32
    bf16 = jnp.bfloat16
    x = x_ref[0]
    oh = oh_ref[0]

    x16 = x.astype(bf16)
    hm_blocks = []
    for rb in range(_I // _RB):
        xb = x[rb * _RB:(rb + 1) * _RB, :]
        xb16 = x16[rb * _RB:(rb + 1) * _RB, :]
        xx = jnp.concatenate(
            [xb16 * xb16[:, c:c + 1] for c in range(_C)], axis=1)
        hm_blocks.append(
            jnp.dot(xx, tfw_ref[...], preferred_element_type=f32)
            + xb @ bfw_ref[...])
    hm = jnp.concatenate(hm_blocks, axis=0)
    hd = hm[:, :_D]
    msg = hm[:, _D:]

    gate = jax.nn.sigmoid(
        jnp.sum(msg * wm_row_ref[...], axis=1, keepdims=True) + bm_ref[...])
    gm = gate * msg
    seg = lax.dot_general(oh, gm, (((0,), (0,)), ((), ())))
    cnt = jnp.sum(oh, axis=0)[:, None]
    agg = seg / jnp.maximum(cnt, 1.0)
    h = hd + oh @ agg

    t = h + jnp.dot(h.astype(bf16), wb_ref[...].astype(bf16),
                    preferred_element_type=f32) + bb_ref[...]
    conv = jnp.maximum(
        jnp.dot(t.astype(bf16), wsig_ref[...].astype(bf16),
                preferred_element_type=f32) + bsig_ref[...], 0.0)

    fm = jnp.concatenate([x, jnp.zeros((_I, _P), f32)], axis=1)
    sg = jnp.sum(fm + conv, axis=0, keepdims=True) * (1.0 / _I)
    a1 = jnp.maximum(sg @ wm1_ref[...] + bm1_ref[...], 0.0)
    a2 = jnp.maximum(sg @ wm2_ref[...] + bm2_ref[...], 0.0)
    mx = jnp.maximum(a1, a2)
    e1 = jnp.exp(a1 - mx)
    e2 = jnp.exp(a2 - mx)
    den = e1 + e2
    fused = (e1 / den) * fm + (e2 / den) * conv
    out_ref[0, :_I, :] = fused

    fused16 = fused.astype(bf16)
    neg = jnp.bfloat16(-jnp.inf)
    enc_rows = []
    for k in range(_K):
        mk = oh[:, k:k + 1] > 0.0
        enc_rows.append(
            jnp.max(jnp.where(mk, fused16, neg), axis=0, keepdims=True))
    enc = jnp.concatenate(enc_rows, axis=0).astype(f32)
    enc = jnp.where(cnt > 0.0, enc, 0.0)

    ups = enc @ wm3_ref[...] + bm3_ref[...]
    for k in range(_K):
        out_ref[0, _I + 2 * k, :] = ups[k, :_D]
        out_ref[0, _I + 2 * k + 1, :] = ups[k, _D:]


def kernel(feature_matrix_batch, Wmlp, bmlp, Wf, bf, Ww, bw, Wm, bm, Wb, bb,
           Wsig, bsig, Wm1, bm1, Wm2, bm2, Wm3, bm3):
    f32 = jnp.float32

    def fixed(shape):
        return pl.BlockSpec(shape, lambda i: (0,) * len(shape))

    x_all, oh_all = pl.pallas_call(
        _labels_body,
        in_specs=[
            pl.BlockSpec((_NCLOUD, _I, _C), lambda: (0, 0, 0)),
            pl.BlockSpec((_C, _C), lambda: (0, 0)),
            pl.BlockSpec((1, _C), lambda: (0, 0)),
        ],
        out_specs=[
            pl.BlockSpec((_NCLOUD, _I, _C), lambda: (0, 0, 0)),
            pl.BlockSpec((_NCLOUD, _I, _K), lambda: (0, 0, 0)),
        ],
        out_shape=[
            jax.ShapeDtypeStruct((_NCLOUD, _I, _C), f32),
            jax.ShapeDtypeStruct((_NCLOUD, _I, _K), f32),
        ],
        scratch_shapes=[pltpu.VMEM((_N, _KG), f32)],
    )(feature_matrix_batch, Wmlp, bmlp.reshape(1, _C))

    tfw = jnp.concatenate(
        [Wf.reshape(_C * _C, _D), Ww.reshape(_C * _C, _D)],
        axis=1).astype(jnp.bfloat16)
    bfw = jnp.concatenate([bf.reshape(_C, _D), bw.reshape(_C, _D)], axis=1)
    args = (x_all, oh_all, tfw, bfw, Wm.reshape(1, _D), bm.reshape(1, 1),
            Wb, bb.reshape(1, _D), Wsig, bsig.reshape(1, _D),
            Wm1, bm1.reshape(1, _D), Wm2, bm2.reshape(1, _D),
            Wm3, bm3.reshape(1, _BRANCH * _D))
    in_specs = [
        pl.BlockSpec((1, _I, _C), lambda i: (i, 0, 0)),
        pl.BlockSpec((1, _I, _K), lambda i: (i, 0, 0)),
        fixed((_C * _C, 2 * _D)), fixed((_C, 2 * _D)),
        fixed((1, _D)), fixed((1, 1)),
        fixed((_D, _D)), fixed((1, _D)),
        fixed((_D, _D)), fixed((1, _D)),
        fixed((_D, _D)), fixed((1, _D)),
        fixed((_D, _D)), fixed((1, _D)),
        fixed((_D, _BRANCH * _D)), fixed((1, _BRANCH * _D)),
    ]
    nrows = _I + _BRANCH * _K
    return pl.pallas_call(
        _cloud_body,
        grid=(_NCLOUD,),
        in_specs=in_specs,
        out_specs=pl.BlockSpec((1, nrows, _D), lambda i: (i, 0, 0)),
        out_shape=jax.ShapeDtypeStruct((_NCLOUD, nrows, _D), f32),
    )(*args)


# speedup vs baseline: 1.0961x; 1.0129x over previous
import jax
import jax.numpy as jnp
from jax import lax
from jax.experimental import pallas as pl
from jax.experimental.pallas import tpu as pltpu

_C = 64
_P = 16
_D = _C + _P
_K = 16
_BRANCH = 2
_NCLOUD = 8
_I = 1024
_LR = 0.001
_RB = 256
_N = _NCLOUD * _I
_KG = _NCLOUD * _K


def _labels_body(fmb_ref, wmlp_ref, bmlp_ref, x_ref, oh_ref, oh_scr):
    f32 = jnp.float32
    bf16 = jnp.bfloat16
    hp = lax.Precision.HIGHEST
    fmb = fmb_ref[...].reshape(_N, _C)
    mm = jnp.dot(fmb.astype(bf16), wmlp_ref[...].astype(bf16),
                 preferred_element_type=f32)
    x2 = fmb + _LR * jnp.maximum(mm + bmlp_ref[...], 0.0)
    x_ref[...] = x2.reshape(_NCLOUD, _I, _C)

    centT0 = jnp.concatenate(
        [jnp.transpose(x2[g * _I:g * _I + _K, :]) for g in range(_NCLOUD)],
        axis=1)
    col = lax.broadcasted_iota(jnp.int32, (_I, _KG), 1)
    colg = col // _K
    big = jnp.float32(1e30)

    def outer(it, centT):
        c2row = jnp.sum(centT * centT, axis=0, keepdims=True)

        def inner(g, carry):
            sat, cntr = carry
            xc = x_ref[g]
            s = jnp.dot(xc, centT, precision=hp)
            d = jnp.where(colg == g, c2row - 2.0 * s, big)
            dmin = jnp.min(d, axis=1, keepdims=True)
            lab = jnp.min(jnp.where(d == dmin, col, _KG), axis=1,
                          keepdims=True)
            ohc = (col == lab).astype(f32)
            oh_scr[pl.ds(g * _I, _I), :] = ohc
            sat = sat + lax.dot_general(
                xc.astype(bf16), ohc.astype(bf16),
                (((0,), (0,)), ((), ())), preferred_element_type=f32)
            cntr = cntr + jnp.sum(ohc, axis=0, keepdims=True)
            return (sat, cntr)

        sat, cntr = lax.fori_loop(
            0, _NCLOUD, inner,
            (jnp.zeros((_C, _KG), f32), jnp.zeros((1, _KG), f32)))
        return jnp.where(cntr > 0.0, sat / jnp.maximum(cntr, 1.0), centT)

    lax.fori_loop(0, 5, outer, centT0)
    for g in range(_NCLOUD):
        oh_ref[g] = oh_scr[g * _I:(g + 1) * _I, g * _K:(g + 1) * _K]


def _cloud_body(x_ref, oh_ref, tfw_ref, bfw_ref, wm_row_ref,
                bm_ref, wb_ref, bb_ref, wsig_ref, bsig_ref, wm1_ref, bm1_ref,
                wm2_ref, bm2_ref, wm3_ref, bm3_ref, out_ref):
    f32 = jnp.float32
    bf16 = jnp.bfloat16
    x = x_ref[0]
    oh = oh_ref[0]

    x16 = x.astype(bf16)
    hm_blocks = []
    for rb in range(_I // _RB):
        xb = x[rb * _RB:(rb + 1) * _RB, :]
        xb16 = x16[rb * _RB:(rb + 1) * _RB, :]
        xx = jnp.concatenate(
            [xb16 * xb16[:, c:c + 1] for c in range(_C)], axis=1)
        hm_blocks.append(
            jnp.dot(xx, tfw_ref[...], preferred_element_type=f32)
            + xb @ bfw_ref[...])
    hm = jnp.concatenate(hm_blocks, axis=0)
    hd = hm[:, :_D]
    msg = hm[:, _D:]

    gate = jax.nn.sigmoid(
        jnp.sum(msg * wm_row_ref[...], axis=1, keepdims=True) + bm_ref[...])
    gm = gate * msg
    seg = lax.dot_general(oh, gm, (((0,), (0,)), ((), ())))
    cnt = jnp.sum(oh, axis=0)[:, None]
    agg = seg / jnp.maximum(cnt, 1.0)
    h = hd + oh @ agg

    t = h + h @ wb_ref[...] + bb_ref[...]
    conv = jnp.maximum(t @ wsig_ref[...] + bsig_ref[...], 0.0)

    fm = jnp.concatenate([x, jnp.zeros((_I, _P), f32)], axis=1)
    sg = jnp.sum(fm + conv, axis=0, keepdims=True) * (1.0 / _I)
    a1 = jnp.maximum(sg @ wm1_ref[...] + bm1_ref[...], 0.0)
    a2 = jnp.maximum(sg @ wm2_ref[...] + bm2_ref[...], 0.0)
    mx = jnp.maximum(a1, a2)
    e1 = jnp.exp(a1 - mx)
    e2 = jnp.exp(a2 - mx)
    den = e1 + e2
    fused = (e1 / den) * fm + (e2 / den) * conv
    out_ref[0, :_I, :] = fused

    neg = jnp.float32(-jnp.inf)
    enc_rows = []
    for k in range(_K):
        mk = oh[:, k:k + 1] > 0.0
        enc_rows.append(
            jnp.max(jnp.where(mk, fused, neg), axis=0, keepdims=True))
    enc = jnp.concatenate(enc_rows, axis=0)
    enc = jnp.where(cnt > 0.0, enc, 0.0)

    ups = enc @ wm3_ref[...] + bm3_ref[...]
    for k in range(_K):
        out_ref[0, _I + 2 * k, :] = ups[k, :_D]
        out_ref[0, _I + 2 * k + 1, :] = ups[k, _D:]


def kernel(feature_matrix_batch, Wmlp, bmlp, Wf, bf, Ww, bw, Wm, bm, Wb, bb,
           Wsig, bsig, Wm1, bm1, Wm2, bm2, Wm3, bm3):
    f32 = jnp.float32

    def fixed(shape):
        return pl.BlockSpec(shape, lambda i: (0,) * len(shape))

    x_all, oh_all = pl.pallas_call(
        _labels_body,
        in_specs=[
            pl.BlockSpec((_NCLOUD, _I, _C), lambda: (0, 0, 0)),
            pl.BlockSpec((_C, _C), lambda: (0, 0)),
            pl.BlockSpec((1, _C), lambda: (0, 0)),
        ],
        out_specs=[
            pl.BlockSpec((_NCLOUD, _I, _C), lambda: (0, 0, 0)),
            pl.BlockSpec((_NCLOUD, _I, _K), lambda: (0, 0, 0)),
        ],
        out_shape=[
            jax.ShapeDtypeStruct((_NCLOUD, _I, _C), f32),
            jax.ShapeDtypeStruct((_NCLOUD, _I, _K), f32),
        ],
        scratch_shapes=[pltpu.VMEM((_N, _KG), f32)],
    )(feature_matrix_batch, Wmlp, bmlp.reshape(1, _C))

    tfw = jnp.concatenate(
        [Wf.reshape(_C * _C, _D), Ww.reshape(_C * _C, _D)],
        axis=1).astype(jnp.bfloat16)
    bfw = jnp.concatenate([bf.reshape(_C, _D), bw.reshape(_C, _D)], axis=1)
    args = (x_all, oh_all, tfw, bfw, Wm.reshape(1, _D), bm.reshape(1, 1),
            Wb, bb.reshape(1, _D), Wsig, bsig.reshape(1, _D),
            Wm1, bm1.reshape(1, _D), Wm2, bm2.reshape(1, _D),
            Wm3, bm3.reshape(1, _BRANCH * _D))
    in_specs = [
        pl.BlockSpec((1, _I, _C), lambda i: (i, 0, 0)),
        pl.BlockSpec((1, _I, _K), lambda i: (i, 0, 0)),
        fixed((_C * _C, 2 * _D)), fixed((_C, 2 * _D)),
        fixed((1, _D)), fixed((1, 1)),
        fixed((_D, _D)), fixed((1, _D)),
        fixed((_D, _D)), fixed((1, _D)),
        fixed((_D, _D)), fixed((1, _D)),
        fixed((_D, _D)), fixed((1, _D)),
        fixed((_D, _BRANCH * _D)), fixed((1, _BRANCH * _D)),
    ]
    nrows = _I + _BRANCH * _K
    return pl.pallas_call(
        _cloud_body,
        grid=(_NCLOUD,),
        in_specs=in_specs,
        out_specs=pl.BlockSpec((1, nrows, _D), lambda i: (i, 0, 0)),
        out_shape=jax.ShapeDtypeStruct((_NCLOUD, nrows, _D), f32),
    )(*args)
